# baseline (speedup 1.0000x reference)
import functools

import jax
import jax.numpy as jnp
from jax import lax
from jax.experimental import pallas as pl
from jax.experimental.pallas import tpu as pltpu
from jax.experimental.pallas import tpu_sc as plsc

N = 10000
E = 320000
H = 64

NC = 2
NS = 16
NW = NC * NS
EW = E // NW
K = 80
NCH = EW // K
NACC = 10112
NPS = NACC // NS
NST = 8
NDR = NPS // NST
HP = 128

_LOG2 = 0.6931471805599453


_LOG2E = 1.4426950408889634


def _ssp(x):
    z = jnp.exp2(jnp.abs(x) * (-_LOG2E))
    return jnp.maximum(x, 0.0) + jnp.log2(1.0 + z) * _LOG2 - _LOG2


def _nodeproj_body(x_ref, w_ref, b_ref, o_ref):
    o_ref[...] = (
        jnp.dot(x_ref[...], w_ref[...], preferred_element_type=jnp.float32)
        + b_ref[...]
    )


def _edgemlp_body(xlo_ref, xhi_ref, w1_ref, b1_ref, w2_ref, b2_ref, o_ref):
    x = jnp.concatenate([xlo_ref[...], xhi_ref[...]], axis=0)
    h = lax.dot_general(x, w1_ref[...], (((0,), (0,)), ((), ())),
                        preferred_element_type=jnp.float32)
    h = _ssp(h + b1_ref[...])
    h = jnp.dot(h, w2_ref[...], preferred_element_type=jnp.float32)
    o_ref[...] = _ssp(h + b2_ref[...])


def _outproj_body(h_ref, w_ref, b_ref, o_ref):
    h = h_ref[0, :, :H] + h_ref[1, :, :H]
    o = jnp.dot(h, w_ref[...], preferred_element_type=jnp.float32)
    o_ref[...] = _ssp(o + b_ref[...])


def _nodeproj(node_feats, W_node, b_node):
    BN = 2000
    Wp = jnp.pad(W_node, ((0, 0), (0, HP - H)))
    bp = jnp.pad(b_node, (0, HP - H)).reshape(1, HP)
    return pl.pallas_call(
        _nodeproj_body,
        grid=(N // BN,),
        in_specs=[
            pl.BlockSpec((BN, 128), lambda i: (i, 0)),
            pl.BlockSpec((128, HP), lambda i: (0, 0)),
            pl.BlockSpec((1, HP), lambda i: (0, 0)),
        ],
        out_specs=pl.BlockSpec((BN, HP), lambda i: (i, 0)),
        out_shape=jax.ShapeDtypeStruct((N, HP), jnp.float32),
    )(node_feats, Wp, bp)


def _edgemlp(edge_feats, W_e1, b_e1, W_e2, b_e2):
    E2 = E // 2
    BE = 16000
    NB = E2 // BE
    W1d = jnp.zeros((32, 2 * H), jnp.float32)
    W1d = W1d.at[:16, :H].set(W_e1).at[16:, H:].set(W_e1)
    W2d = jnp.zeros((2 * H, 2 * H), jnp.float32)
    W2d = W2d.at[:H, :H].set(W_e2).at[H:, H:].set(W_e2)
    b1d = jnp.concatenate([b_e1, b_e1]).reshape(1, 2 * H)
    b2d = jnp.concatenate([b_e2, b_e2]).reshape(1, 2 * H)
    return pl.pallas_call(
        _edgemlp_body,
        grid=(NB,),
        in_specs=[
            pl.BlockSpec((16, BE), lambda i: (0, i)),
            pl.BlockSpec((16, BE), lambda i: (0, i + NB)),
            pl.BlockSpec((32, 2 * H), lambda i: (0, 0)),
            pl.BlockSpec((1, 2 * H), lambda i: (0, 0)),
            pl.BlockSpec((2 * H, 2 * H), lambda i: (0, 0)),
            pl.BlockSpec((1, 2 * H), lambda i: (0, 0)),
        ],
        out_specs=pl.BlockSpec((BE, 2 * H), lambda i: (i, 0)),
        out_shape=jax.ShapeDtypeStruct((E2, 2 * H), jnp.float32),
    )(edge_feats.T, edge_feats.T, W1d, b1d, W2d, b2d)


def _outproj(partials, W_o, b_o):
    BN = 2000
    return pl.pallas_call(
        _outproj_body,
        grid=(N // BN,),
        in_specs=[
            pl.BlockSpec((2, BN, HP), lambda i: (0, i, 0)),
            pl.BlockSpec((H, 128), lambda i: (0, 0)),
            pl.BlockSpec((1, 128), lambda i: (0, 0)),
        ],
        out_specs=pl.BlockSpec((BN, 128), lambda i: (i, 0)),
        out_shape=jax.ShapeDtypeStruct((N, 128), jnp.float32),
    )(partials, W_o, b_o.reshape(1, 128))


@functools.partial(
    pl.kernel,
    out_type=jax.ShapeDtypeStruct((NC, NACC, HP), jnp.float32),
    mesh=plsc.VectorSubcoreMesh(core_axis_name="c", subcore_axis_name="s"),
    scratch_types=[
        pltpu.VMEM_SHARED((NACC, HP), jnp.float32),
        pltpu.VMEM((32, K), jnp.int32),
        pltpu.VMEM((32, K), jnp.int32),
        pltpu.VMEM((2, K // 2, HP), jnp.float32),
        pltpu.VMEM((2, K, HP), jnp.float32),
        pltpu.VMEM((K, HP), jnp.float32),
        pltpu.SemaphoreType.DMA,
        pltpu.SemaphoreType.DMA,
        pltpu.SemaphoreType.DMA,
        pltpu.SemaphoreType.DMA,
        pltpu.SemaphoreType.DMA,
    ],
)
def _sc_aggregate(hv, he2, src3, dst3, out, h_sh, src_v, dst_v, he_b, g_b,
                  p_b, s_he0, s_he1, s_g0, s_g1, s_sc):
    c = lax.axis_index("c")
    s = lax.axis_index("s")
    wid = c * NS + s
    s_he = (s_he0, s_he1)
    s_g = (s_g0, s_g1)

    IB = 16

    def idx_row(j):
        return ((j // IB) % 2) * IB + j % IB

    def load_idx_block(j):
        off = pl.multiple_of((j // IB) * IB, 8)
        p8 = ((j // IB) % 2) * IB
        pltpu.sync_copy(src3.at[wid, pl.ds(off, IB)],
                        src_v.at[pl.ds(p8, IB)])
        pltpu.sync_copy(dst3.at[wid, pl.ds(off, IB)],
                        dst_v.at[pl.ds(p8, IB)])

    def he_rows(j):
        off = pl.multiple_of(wid * (EW // 2) + j * (K // 2), 8)
        return he2.at[pl.ds(off, K // 2)]

    def issue_loads(j, b):
        pltpu.async_copy(he_rows(j), he_b.at[b], s_he[b])
        pltpu.async_copy(hv.at[src_v.at[idx_row(j)]], g_b.at[b], s_g[b])

    def wait_loads(j, b):
        pltpu.make_async_copy(he_rows(j), he_b.at[b], s_he[b]).wait()
        pltpu.make_async_copy(
            hv.at[src_v.at[idx_row(j)]], g_b.at[b], s_g[b]).wait()

    def issue_scatter(j):
        pltpu.async_copy(p_b, h_sh.at[dst_v.at[idx_row(j)]], s_sc, add=True)

    def wait_scatter(j):
        pltpu.make_async_copy(p_b, h_sh.at[dst_v.at[idx_row(j)]], s_sc).wait()

    def mul(b):
        def mrow(r2):
            for q in range(HP // 16):
                row = 2 * r2 + q // 4
                d = pl.ds((q % 4) * 16, 16)
                p_b[row, d] = g_b[b, row, d] * he_b[b, r2, pl.ds(q * 16, 16)]

        plsc.parallel_loop(0, K // 2, unroll=8)(mrow)

    def zrow(r, carry):
        for q in range(HP // 16):
            p_b[r, pl.ds(q * 16, 16)] = jnp.zeros((16,), jnp.float32)
        return carry

    lax.fori_loop(0, NST, zrow, 0)

    def zissue(p, carry):
        pltpu.async_copy(p_b.at[pl.ds(0, NST)],
                         h_sh.at[pl.ds(s * NPS + p * NST, NST)], s_sc)
        return carry

    lax.fori_loop(0, NDR, zissue, 0)

    def zwait(p, carry):
        pltpu.make_async_copy(
            p_b.at[pl.ds(0, NST)],
            h_sh.at[pl.ds(s * NPS + p * NST, NST)], s_sc).wait()
        return carry

    lax.fori_loop(0, NDR, zwait, 0)

    load_idx_block(0)
    issue_loads(0, 0)
    issue_loads(1, 1)
    plsc.subcore_barrier()

    def half(j, b, t):
        wait_loads(j, b)
        if t is None:
            wait_scatter(j - 1)
        else:
            @pl.when(t > 0)
            def _():
                wait_scatter(j - 1)
        mul(b)
        issue_scatter(j)
        jn = j + 2

        @pl.when(jn < NCH)
        def _():
            @pl.when(jn % 16 == 0)
            def _():
                load_idx_block(jn)
            issue_loads(jn, b)

    def pair(t, carry):
        half(2 * t, 0, t)
        half(2 * t + 1, 1, None)
        return carry

    lax.fori_loop(0, NCH // 2, pair, 0)
    wait_loads(NCH - 1, 0)
    wait_scatter(NCH - 2)
    mul(0)
    issue_scatter(NCH - 1)
    wait_scatter(NCH - 1)
    plsc.subcore_barrier()

    pltpu.sync_copy(h_sh.at[pl.ds(s * NPS, NPS)],
                    out.at[c, pl.ds(s * NPS, NPS)])


def _interleave_idx(idx):
    E2 = E // 2
    a = idx[:E2].reshape(NW, NCH, K // 2)
    b = idx[E2:].reshape(NW, NCH, K // 2)
    x = jnp.stack([a, b], axis=-1).reshape(NW, NCH, K)
    return jnp.pad(x, ((0, 0), (0, 128 - NCH), (0, 0)))


def kernel(node_feats, edge_feats, edge_index, W_node, b_node, W_e1, b_e1,
           W_e2, b_e2, W_o, b_o):
    src3 = _interleave_idx(edge_index[0].astype(jnp.int32))
    dst3 = _interleave_idx(edge_index[1].astype(jnp.int32))
    hv = _nodeproj(node_feats, W_node, b_node)
    he2 = _edgemlp(edge_feats, W_e1, b_e1, W_e2, b_e2)
    partials = _sc_aggregate(hv, he2, src3, dst3)
    return _outproj(partials, W_o, b_o)


# speedup vs baseline: 1.0131x; 1.0131x over previous
import functools

import jax
import jax.numpy as jnp
from jax import lax
from jax.experimental import pallas as pl
from jax.experimental.pallas import tpu as pltpu
from jax.experimental.pallas import tpu_sc as plsc

N = 10000
E = 320000
H = 64

NC = 2
NS = 16
NW = NC * NS
EW = E // NW
K = 80
NCH = EW // K
NACC = 10112
NPS = NACC // NS
NST = 8
NDR = NPS // NST
HP = 128

_LOG2 = 0.6931471805599453


_LOG2E = 1.4426950408889634


def _ssp(x):
    z = jnp.exp2(jnp.abs(x) * (-_LOG2E))
    return jnp.maximum(x, 0.0) + jnp.log2(1.0 + z) * _LOG2 - _LOG2


def _nodeproj_body(x_ref, w_ref, b_ref, o_ref):
    o_ref[...] = (
        jnp.dot(x_ref[...], w_ref[...], preferred_element_type=jnp.float32)
        + b_ref[...]
    )


def _edgemlp_body(xlo_ref, xhi_ref, w1_ref, b1_ref, w2_ref, b2_ref, o_ref):
    x = jnp.concatenate([xlo_ref[...], xhi_ref[...]], axis=0)
    h = lax.dot_general(x, w1_ref[...], (((0,), (0,)), ((), ())),
                        preferred_element_type=jnp.float32)
    h = _ssp(h + b1_ref[...])
    h = jnp.dot(h, w2_ref[...], preferred_element_type=jnp.float32)
    o_ref[...] = _ssp(h + b2_ref[...])


def _outproj_body(h_ref, w_ref, b_ref, o_ref):
    h = h_ref[0, :, :H] + h_ref[1, :, :H]
    o = jnp.dot(h, w_ref[...], preferred_element_type=jnp.float32)
    o_ref[...] = _ssp(o + b_ref[...])


def _nodeproj(node_feats, W_node, b_node):
    BN = 2000
    Wp = jnp.pad(W_node, ((0, 0), (0, HP - H)))
    bp = jnp.pad(b_node, (0, HP - H)).reshape(1, HP)
    return pl.pallas_call(
        _nodeproj_body,
        grid=(N // BN,),
        in_specs=[
            pl.BlockSpec((BN, 128), lambda i: (i, 0)),
            pl.BlockSpec((128, HP), lambda i: (0, 0)),
            pl.BlockSpec((1, HP), lambda i: (0, 0)),
        ],
        out_specs=pl.BlockSpec((BN, HP), lambda i: (i, 0)),
        out_shape=jax.ShapeDtypeStruct((N, HP), jnp.float32),
    )(node_feats, Wp, bp)


def _edgemlp(edge_feats, W_e1, b_e1, W_e2, b_e2):
    E2 = E // 2
    BE = 16000
    NB = E2 // BE
    W1d = jnp.zeros((32, 2 * H), jnp.float32)
    W1d = W1d.at[:16, :H].set(W_e1).at[16:, H:].set(W_e1)
    W2d = jnp.zeros((2 * H, 2 * H), jnp.float32)
    W2d = W2d.at[:H, :H].set(W_e2).at[H:, H:].set(W_e2)
    b1d = jnp.concatenate([b_e1, b_e1]).reshape(1, 2 * H)
    b2d = jnp.concatenate([b_e2, b_e2]).reshape(1, 2 * H)
    return pl.pallas_call(
        _edgemlp_body,
        grid=(NB,),
        in_specs=[
            pl.BlockSpec((16, BE), lambda i: (0, i)),
            pl.BlockSpec((16, BE), lambda i: (0, i + NB)),
            pl.BlockSpec((32, 2 * H), lambda i: (0, 0)),
            pl.BlockSpec((1, 2 * H), lambda i: (0, 0)),
            pl.BlockSpec((2 * H, 2 * H), lambda i: (0, 0)),
            pl.BlockSpec((1, 2 * H), lambda i: (0, 0)),
        ],
        out_specs=pl.BlockSpec((BE, 2 * H), lambda i: (i, 0)),
        out_shape=jax.ShapeDtypeStruct((E2, 2 * H), jnp.float32),
    )(edge_feats.T, edge_feats.T, W1d, b1d, W2d, b2d)


def _outproj(partials, W_o, b_o):
    BN = 2000
    return pl.pallas_call(
        _outproj_body,
        grid=(N // BN,),
        in_specs=[
            pl.BlockSpec((2, BN, HP), lambda i: (0, i, 0)),
            pl.BlockSpec((H, 128), lambda i: (0, 0)),
            pl.BlockSpec((1, 128), lambda i: (0, 0)),
        ],
        out_specs=pl.BlockSpec((BN, 128), lambda i: (i, 0)),
        out_shape=jax.ShapeDtypeStruct((N, 128), jnp.float32),
    )(partials, W_o, b_o.reshape(1, 128))


@functools.partial(
    pl.kernel,
    out_type=jax.ShapeDtypeStruct((NC, NACC, HP), jnp.float32),
    mesh=plsc.VectorSubcoreMesh(core_axis_name="c", subcore_axis_name="s"),
    scratch_types=[
        pltpu.VMEM_SHARED((NACC, HP), jnp.float32),
        pltpu.VMEM((16, K), jnp.int32),
        pltpu.VMEM((16, K), jnp.int32),
        pltpu.VMEM((2, K // 2, HP), jnp.float32),
        pltpu.VMEM((2, K, HP), jnp.float32),
        pltpu.VMEM((K, HP), jnp.float32),
        pltpu.SemaphoreType.DMA,
        pltpu.SemaphoreType.DMA,
        pltpu.SemaphoreType.DMA,
        pltpu.SemaphoreType.DMA,
        pltpu.SemaphoreType.DMA,
    ],
)
def _sc_aggregate(hv, he2, src3, dst3, out, h_sh, src_v, dst_v, he_b, g_b,
                  p_b, s_he0, s_he1, s_g0, s_g1, s_sc):
    c = lax.axis_index("c")
    s = lax.axis_index("s")
    wid = c * NS + s
    s_he = (s_he0, s_he1)
    s_g = (s_g0, s_g1)

    def idx_row(j):
        return ((j // 8) % 2) * 8 + j % 8

    def load_idx_block(j):
        off = pl.multiple_of((j // 8) * 8, 8)
        p8 = ((j // 8) % 2) * 8
        pltpu.sync_copy(src3.at[wid, pl.ds(off, 8)], src_v.at[pl.ds(p8, 8)])
        pltpu.sync_copy(dst3.at[wid, pl.ds(off, 8)], dst_v.at[pl.ds(p8, 8)])

    def he_rows(j):
        off = pl.multiple_of(wid * (EW // 2) + j * (K // 2), 8)
        return he2.at[pl.ds(off, K // 2)]

    def issue_loads(j, b):
        pltpu.async_copy(he_rows(j), he_b.at[b], s_he[b])
        pltpu.async_copy(hv.at[src_v.at[idx_row(j)]], g_b.at[b], s_g[b])

    def wait_loads(j, b):
        pltpu.make_async_copy(he_rows(j), he_b.at[b], s_he[b]).wait()
        pltpu.make_async_copy(
            hv.at[src_v.at[idx_row(j)]], g_b.at[b], s_g[b]).wait()

    def issue_scatter(j):
        pltpu.async_copy(p_b, h_sh.at[dst_v.at[idx_row(j)]], s_sc, add=True)

    def wait_scatter(j):
        pltpu.make_async_copy(p_b, h_sh.at[dst_v.at[idx_row(j)]], s_sc).wait()

    def mul(b):
        def mrow(r2):
            for q in range(HP // 16):
                row = 2 * r2 + q // 4
                d = pl.ds((q % 4) * 16, 16)
                p_b[row, d] = g_b[b, row, d] * he_b[b, r2, pl.ds(q * 16, 16)]

        plsc.parallel_loop(0, K // 2, unroll=4)(mrow)

    def zrow(r, carry):
        for q in range(HP // 16):
            p_b[r, pl.ds(q * 16, 16)] = jnp.zeros((16,), jnp.float32)
        return carry

    lax.fori_loop(0, NST, zrow, 0)

    def zissue(p, carry):
        pltpu.async_copy(p_b.at[pl.ds(0, NST)],
                         h_sh.at[pl.ds(s * NPS + p * NST, NST)], s_sc)
        return carry

    lax.fori_loop(0, NDR, zissue, 0)

    def zwait(p, carry):
        pltpu.make_async_copy(
            p_b.at[pl.ds(0, NST)],
            h_sh.at[pl.ds(s * NPS + p * NST, NST)], s_sc).wait()
        return carry

    lax.fori_loop(0, NDR, zwait, 0)

    load_idx_block(0)
    issue_loads(0, 0)
    issue_loads(1, 1)
    plsc.subcore_barrier()

    def half(j, b, t):
        wait_loads(j, b)
        if t is None:
            wait_scatter(j - 1)
        else:
            @pl.when(t > 0)
            def _():
                wait_scatter(j - 1)
        mul(b)
        issue_scatter(j)
        jn = j + 2

        @pl.when(jn < NCH)
        def _():
            @pl.when(jn % 8 == 0)
            def _():
                load_idx_block(jn)
            issue_loads(jn, b)

    def pair(t, carry):
        half(2 * t, 0, t)
        half(2 * t + 1, 1, None)
        return carry

    lax.fori_loop(0, NCH // 2, pair, 0)
    wait_loads(NCH - 1, 0)
    wait_scatter(NCH - 2)
    mul(0)
    issue_scatter(NCH - 1)
    wait_scatter(NCH - 1)
    plsc.subcore_barrier()

    pltpu.sync_copy(h_sh.at[pl.ds(s * NPS, NPS)],
                    out.at[c, pl.ds(s * NPS, NPS)])


def _interleave_idx(idx):
    E2 = E // 2
    a = idx[:E2].reshape(NW, NCH, K // 2)
    b = idx[E2:].reshape(NW, NCH, K // 2)
    return jnp.stack([a, b], axis=-1).reshape(NW, NCH, K)


def kernel(node_feats, edge_feats, edge_index, W_node, b_node, W_e1, b_e1,
           W_e2, b_e2, W_o, b_o):
    src3 = _interleave_idx(edge_index[0].astype(jnp.int32))
    dst3 = _interleave_idx(edge_index[1].astype(jnp.int32))
    hv = _nodeproj(node_feats, W_node, b_node)
    he2 = _edgemlp(edge_feats, W_e1, b_e1, W_e2, b_e2)
    partials = _sc_aggregate(hv, he2, src3, dst3)
    return _outproj(partials, W_o, b_o)
